# Initial kernel scaffold; baseline (speedup 1.0000x reference)
import jax
import jax.numpy as jnp
from jax.experimental import pallas as pl


def kernel(ids, table):
    raise NotImplementedError("write your pallas kernel here")


# speedup vs baseline: 3.2087x; 3.2087x over previous
import functools

import jax
import jax.numpy as jnp
from jax import lax
from jax.experimental import pallas as pl
from jax.experimental.pallas import tpu as pltpu
from jax.experimental.pallas import tpu_sc as plsc

NC = 2
NS = 16
NW = NC * NS
CHUNK = 128


@functools.partial(jax.jit, static_argnames=())
def _gather_rows(flat_idx, table):
    _, n_chunks, chunk = flat_idx.shape
    n_total = NW * n_chunks
    _, d = table.shape
    mesh = plsc.VectorSubcoreMesh(core_axis_name="c", subcore_axis_name="s")

    @functools.partial(
        pl.kernel,
        out_type=jax.ShapeDtypeStruct((n_total * chunk, d), jnp.float32),
        mesh=mesh,
        scratch_types=[
            pltpu.VMEM((n_chunks, chunk), jnp.int32),
            pltpu.VMEM((chunk, d), jnp.float32),
            pltpu.SemaphoreType.DMA,
        ],
        compiler_params=pltpu.CompilerParams(use_tc_tiling_on_sc=False),
    )
    def body(idx_hbm, table_hbm, out_hbm, idx_v, rows_v, gsem):
        wid = lax.axis_index("s") * NC + lax.axis_index("c")
        pltpu.sync_copy(idx_hbm.at[wid], idx_v)

        def step(j, carry):
            pltpu.async_copy(table_hbm.at[idx_v.at[j]], rows_v, gsem).wait()
            base = (wid * n_chunks + j) * chunk
            pltpu.sync_copy(rows_v, out_hbm.at[pl.ds(base, chunk)])
            return carry

        lax.fori_loop(0, n_chunks, step, 0)

    return body(flat_idx, table)


def kernel(ids, table):
    b, s = ids.shape
    _, d = table.shape
    n = b * s
    flat = ids.reshape(NW, n // (NW * CHUNK), CHUNK)
    out = _gather_rows(flat, table)
    return out.reshape(b, s, d)


# speedup vs baseline: 3.3572x; 1.0463x over previous
import functools

import jax
import jax.numpy as jnp
from jax import lax
from jax.experimental import pallas as pl
from jax.experimental.pallas import tpu as pltpu
from jax.experimental.pallas import tpu_sc as plsc

NC = 2
NS = 16
NW = NC * NS
RPW = 8


def _gather_rows(ids, table):
    b, s = ids.shape
    v, d = table.shape
    rows_w = b // NW
    grp = 4
    n_grp = rows_w // grp
    mesh = plsc.VectorSubcoreMesh(core_axis_name="c", subcore_axis_name="s")

    @functools.partial(
        pl.kernel,
        out_type=jax.ShapeDtypeStruct((b * s, d), jnp.float32),
        mesh=mesh,
        scratch_types=[
            pltpu.VMEM((rows_w, s), jnp.int32),
            pltpu.VMEM((4, grp * s, d), jnp.float32),
            pltpu.SemaphoreType.DMA,
            [pltpu.SemaphoreType.DMA] * 4,
        ],
        compiler_params=pltpu.CompilerParams(use_tc_tiling_on_sc=False),
    )
    def body(ids_hbm, table_hbm, out_hbm, idx_v, rows_v, gsem, osems):
        wid = lax.axis_index("s") * NC + lax.axis_index("c")
        base_row = wid * rows_w
        pltpu.sync_copy(ids_hbm.at[pl.ds(base_row, rows_w)], idx_v)

        def issue_group(g, k):
            for q in range(grp):
                pltpu.async_copy(
                    table_hbm.at[idx_v.at[g * grp + q]],
                    rows_v.at[k].at[pl.ds(q * s, s)],
                    gsem,
                )

        issue_group(0, 0)

        def step(p, carry):
            for k in range(4):
                g = p * 4 + k
                nk = (k + 1) % 4

                @pl.when(g + 1 < n_grp)
                def _():
                    @pl.when(g >= 3)
                    def _():
                        pltpu.make_async_copy(
                            rows_v.at[nk], out_hbm.at[pl.ds(0, grp * s)], osems[nk]
                        ).wait()

                    issue_group(g + 1, nk)

                for q in range(grp):
                    pltpu.make_async_copy(
                        table_hbm.at[idx_v.at[0]],
                        rows_v.at[k].at[pl.ds(q * s, s)],
                        gsem,
                    ).wait()

                pltpu.async_copy(
                    rows_v.at[k],
                    out_hbm.at[pl.ds((base_row + g * grp) * s, grp * s)],
                    osems[k],
                )
            return carry

        lax.fori_loop(0, n_grp // 4, step, 0)
        for k in range(4):
            pltpu.make_async_copy(
                rows_v.at[k], out_hbm.at[pl.ds(0, grp * s)], osems[k]
            ).wait()

    return body(ids, table)


_gather_jit = jax.jit(_gather_rows)


def kernel(ids, table):
    b, s = ids.shape
    _, d = table.shape
    out = _gather_jit(ids, table)
    return out.reshape(b, s, d)
